# baseline (speedup 1.0000x reference)
import functools

import jax
import jax.numpy as jnp
from jax import lax
from jax.experimental import pallas as pl
from jax.experimental.pallas import tpu as pltpu
from jax.experimental.pallas import tpu_sc as plsc

VOCAB = 1000000
EMBED = 64
CTX = 200
BATCH = 4096
SEQ = 200

N_WORKERS = 32
NK = EMBED // 16

TC = 320
TPITCH = 325
A_WORKERS = 25
A_BLOCKS = VOCAB // A_WORKERS // TC

BW = BATCH // N_WORKERS
NS = 2
B_IT = SEQ // NS
OPITCH = 133


def _worker_id():
    return lax.axis_index("s") * 2 + lax.axis_index("c")


def _iota16():
    return lax.iota(jnp.int32, 16)


def _body_a(tt_hbm, tbl_hbm, in0, in1, ob0, ob1, isem0, isem1, ssem0, ssem1):
    wid = _worker_id()
    c0 = wid * (A_BLOCKS * TC)

    def fire_in(i, inb, isem):
        pltpu.async_copy(
            tt_hbm.at[:, pl.ds(c0 + i * TC, TC)], inb.at[:, pl.ds(0, TC)], isem
        )

    def drain_in(inb, isem):
        pltpu.make_async_copy(
            tt_hbm.at[:, pl.ds(0, TC)], inb.at[:, pl.ds(0, TC)], isem
        ).wait()

    def fire_store(i, ob, ssem):
        pltpu.async_copy(ob, tbl_hbm.at[pl.ds(c0 + i * TC, TC)], ssem)

    def wait_store(ob, ssem):
        pltpu.make_async_copy(ob, tbl_hbm.at[pl.ds(0, TC)], ssem).wait()

    def transpose(inb, ob):
        rows = [_iota16() + 16 * k for k in range(NK)]

        @plsc.parallel_loop(0, TC, 1, unroll=2)
        def _t(t):
            tsplat = jnp.full((16,), t, jnp.int32)
            for k in range(NK):
                ob[t, pl.ds(16 * k, 16)] = plsc.load_gather(inb, [rows[k], tsplat])

    s0 = (in0, ob0, isem0, ssem0)
    s1 = (in1, ob1, isem1, ssem1)

    def steady(i, X, Y):
        (inX, obX, isemX, ssemX) = X
        (inY, obY, isemY, ssemY) = Y
        wait_store(obY, ssemY)
        fire_in(i + 1, inY, isemY)
        drain_in(inX, isemX)
        transpose(inX, obX)
        fire_store(i, obX, ssemX)

    @pl.when(wid < A_WORKERS)
    def _active():
        fire_in(0, in0, isem0)
        drain_in(in0, isem0)
        fire_in(1, in1, isem1)
        transpose(in0, ob0)
        fire_store(0, ob0, ssem0)

        def step(i, _):
            @pl.when(lax.rem(i, 2) == 1)
            def _odd():
                steady(i, s1, s0)

            @pl.when(lax.rem(i, 2) == 0)
            def _even():
                steady(i, s0, s1)

            return 0

        lax.fori_loop(1, A_BLOCKS - 1, step, 0)

        wait_store(ob1, ssem1)
        drain_in(in0, isem0)
        transpose(in0, ob0)
        fire_store(A_BLOCKS - 1, ob0, ssem0)
        wait_store(ob0, ssem0)


def _body_b(idx_hbm, table_hbm, pos_hbm, out_hbm, pos_v,
            idx0, buf0, obuf0, idx1, buf1, obuf1,
            gsem0, gsem1, ssem0, ssem1, isem0, isem1):
    wid = _worker_id()
    bw0 = wid * BW

    pltpu.sync_copy(pos_hbm, pos_v)

    def fire_idx(i, idx, isem):
        s = jnp.minimum(i, B_IT - 1) * NS
        pltpu.async_copy(idx_hbm.at[pl.ds(s, NS), pl.ds(bw0, BW)], idx, isem)

    def wait_idx(idx, isem):
        pltpu.make_async_copy(
            idx_hbm.at[pl.ds(0, NS), pl.ds(bw0, BW)], idx, isem
        ).wait()

    def fire_gathers(idx, buf, gsem):
        for j in range(NS):
            pltpu.async_copy(table_hbm.at[idx.at[j]], buf.at[j], gsem)

    def drain_gathers(idx, buf, gsem):
        for j in range(NS):
            pltpu.make_async_copy(table_hbm.at[idx.at[j]], buf.at[j], gsem).wait()

    def compute(i, buf, obuf):
        iota = _iota16()
        for j in range(NS):
            pbase = (i * NS + j) * EMBED
            pv = [pos_v[pl.ds(pbase + 16 * k, 16)] for k in range(NK)]
            jsplat = jnp.full((16,), j, jnp.int32)
            drows = [iota + 16 * k for k in range(NK)]

            @plsc.parallel_loop(0, BW, 1, unroll=2)
            def _b(b):
                bsplat = jnp.full((16,), b, jnp.int32)
                for k in range(NK):
                    v = buf[j, b, pl.ds(16 * k, 16)] + pv[k]
                    plsc.store_scatter(obuf, [jsplat, drows[k], bsplat], v)

    def fire_store(i, obuf, ssem):
        pltpu.async_copy(
            obuf.at[:, :, pl.ds(0, BW)],
            out_hbm.at[pl.ds(i * NS, NS), :, pl.ds(bw0, BW)],
            ssem,
        )

    def wait_store(obuf, ssem):
        pltpu.make_async_copy(
            obuf.at[:, :, pl.ds(0, BW)],
            out_hbm.at[pl.ds(0, NS), :, pl.ds(bw0, BW)],
            ssem,
        ).wait()

    s0 = (idx0, buf0, obuf0, gsem0, ssem0, isem0)
    s1 = (idx1, buf1, obuf1, gsem1, ssem1, isem1)

    def steady(i, X, Y):
        (idxX, bufX, obufX, gsemX, ssemX, isemX) = X
        (idxY, bufY, obufY, gsemY, ssemY, isemY) = Y
        wait_store(obufY, ssemY)
        wait_idx(idxY, isemY)
        fire_gathers(idxY, bufY, gsemY)
        drain_gathers(idxX, bufX, gsemX)
        fire_idx(i + 2, idxX, isemX)
        compute(i, bufX, obufX)
        fire_store(i, obufX, ssemX)

    fire_idx(0, idx0, isem0)
    wait_idx(idx0, isem0)
    fire_gathers(idx0, buf0, gsem0)
    fire_idx(1, idx1, isem1)

    wait_idx(idx1, isem1)
    fire_gathers(idx1, buf1, gsem1)
    drain_gathers(idx0, buf0, gsem0)
    fire_idx(2, idx0, isem0)
    compute(0, buf0, obuf0)
    fire_store(0, obuf0, ssem0)

    def pair(t, _):
        i = 2 * t + 1
        steady(i, s1, s0)
        steady(i + 1, s0, s1)
        return 0

    lax.fori_loop(0, (B_IT - 2) // 2, pair, 0)

    wait_store(obuf0, ssem0)
    drain_gathers(idx1, buf1, gsem1)
    compute(B_IT - 1, buf1, obuf1)
    fire_store(B_IT - 1, obuf1, ssem1)
    wait_idx(idx0, isem0)
    wait_store(obuf1, ssem1)


@jax.jit
def kernel(token_ids, token_emb, pos_emb):
    mesh = plsc.VectorSubcoreMesh(core_axis_name="c", subcore_axis_name="s")
    params = pltpu.CompilerParams(
        use_tc_tiling_on_sc=False, needs_layout_passes=False
    )

    tbl = pl.kernel(
        _body_a,
        out_type=jax.ShapeDtypeStruct((VOCAB, EMBED), jnp.float32),
        mesh=mesh,
        compiler_params=params,
        scratch_types=[
            pltpu.VMEM((EMBED, TPITCH), jnp.float32),
            pltpu.VMEM((EMBED, TPITCH), jnp.float32),
            pltpu.VMEM((TC, EMBED), jnp.float32),
            pltpu.VMEM((TC, EMBED), jnp.float32),
            pltpu.SemaphoreType.DMA,
            pltpu.SemaphoreType.DMA,
            pltpu.SemaphoreType.DMA,
            pltpu.SemaphoreType.DMA,
        ],
    )(token_emb.T)

    def slot_scratch():
        return [
            pltpu.VMEM((NS, BW), jnp.int32),
            pltpu.VMEM((NS, BW, EMBED), jnp.float32),
            pltpu.VMEM((NS, EMBED, OPITCH), jnp.float32),
        ]

    out = pl.kernel(
        _body_b,
        out_type=jax.ShapeDtypeStruct((SEQ, EMBED, BATCH), jnp.float32),
        mesh=mesh,
        compiler_params=params,
        scratch_types=[
            pltpu.VMEM((CTX * EMBED,), jnp.float32),
            *slot_scratch(),
            *slot_scratch(),
            pltpu.SemaphoreType.DMA,
            pltpu.SemaphoreType.DMA,
            pltpu.SemaphoreType.DMA,
            pltpu.SemaphoreType.DMA,
            pltpu.SemaphoreType.DMA,
            pltpu.SemaphoreType.DMA,
        ],
    )(
        token_ids.T.astype(jnp.int32),
        tbl,
        pos_emb.reshape(CTX * EMBED),
    )
    return out.transpose(2, 0, 1)


# speedup vs baseline: 5.3102x; 5.3102x over previous
import functools

import jax
import jax.numpy as jnp
from jax import lax
from jax.experimental import pallas as pl
from jax.experimental.pallas import tpu as pltpu
from jax.experimental.pallas import tpu_sc as plsc

VOCAB = 1000000
EMBED = 64
CTX = 200
BATCH = 4096
SEQ = 200

N_WORKERS = 32
NK = EMBED // 16
BW = BATCH // N_WORKERS
NS = 2
N_IT = SEQ // NS
OPITCH = 133


def _worker_id():
    return lax.axis_index("s") * 2 + lax.axis_index("c")


def _body(idx_hbm, table_hbm, pos_hbm, out_hbm, pos_v,
          idx0, gidx0, poff0, buf0, obuf0,
          idx1, gidx1, poff1, buf1, obuf1,
          gsem0, gsem1, ssem0, ssem1, isem0, isem1):
    wid = _worker_id()
    bw0 = wid * BW

    pltpu.sync_copy(pos_hbm, pos_v)

    def fire_idx(i, idx, isem):
        s = jnp.minimum(i, N_IT - 1) * NS
        pltpu.async_copy(idx_hbm.at[pl.ds(s, NS), pl.ds(bw0, BW)], idx, isem)

    def wait_idx(idx, isem):
        pltpu.make_async_copy(
            idx_hbm.at[pl.ds(0, NS), pl.ds(bw0, BW)], idx, isem
        ).wait()

    def prep_gather(idx, gidx, poff):
        for j in range(NS):
            for c in range(BW // 16):
                sl = pl.ds(c * 16, 16)
                v = idx[j, sl]
                gidx[j, sl] = lax.shift_right_logical(v, 1)
                poff[j, sl] = lax.shift_left(v & 1, 6)

    def fire_gathers(gidx, buf, gsem):
        for j in range(NS):
            pltpu.async_copy(table_hbm.at[gidx.at[j]], buf.at[j], gsem)

    def drain_gathers(gidx, buf, gsem):
        for j in range(NS):
            pltpu.make_async_copy(table_hbm.at[gidx.at[j]], buf.at[j], gsem).wait()

    def compute(i, poff, buf, obuf):
        iota = lax.iota(jnp.int32, 16)
        drows = [iota + 16 * k for k in range(NK)]
        for j in range(NS):
            pbase = (i * NS + j) * EMBED
            pv = [pos_v[pl.ds(pbase + 16 * k, 16)] for k in range(NK)]
            jsplat = jnp.full((16,), j, jnp.int32)

            @plsc.parallel_loop(0, BW, 1, unroll=2)
            def _b(b):
                bsplat = jnp.full((16,), b, jnp.int32)
                pofb = plsc.load_gather(poff, [jsplat, bsplat])
                for k in range(NK):
                    v = plsc.load_gather(buf, [jsplat, bsplat, pofb + drows[k]])
                    plsc.store_scatter(obuf, [jsplat, drows[k], bsplat], v + pv[k])

    def fire_store(i, obuf, ssem):
        pltpu.async_copy(
            obuf.at[:, :, pl.ds(0, BW)],
            out_hbm.at[pl.ds(i * NS, NS), :, pl.ds(bw0, BW)],
            ssem,
        )

    def wait_store(obuf, ssem):
        pltpu.make_async_copy(
            obuf.at[:, :, pl.ds(0, BW)],
            out_hbm.at[pl.ds(0, NS), :, pl.ds(bw0, BW)],
            ssem,
        ).wait()

    s0 = (idx0, gidx0, poff0, buf0, obuf0, gsem0, ssem0, isem0)
    s1 = (idx1, gidx1, poff1, buf1, obuf1, gsem1, ssem1, isem1)

    def steady(i, X, Y):
        (idxX, gidxX, poffX, bufX, obufX, gsemX, ssemX, isemX) = X
        (idxY, gidxY, poffY, bufY, obufY, gsemY, ssemY, isemY) = Y
        wait_store(obufY, ssemY)
        wait_idx(idxY, isemY)
        prep_gather(idxY, gidxY, poffY)
        fire_gathers(gidxY, bufY, gsemY)
        drain_gathers(gidxX, bufX, gsemX)
        fire_idx(i + 2, idxX, isemX)
        compute(i, poffX, bufX, obufX)
        fire_store(i, obufX, ssemX)

    fire_idx(0, idx0, isem0)
    wait_idx(idx0, isem0)
    prep_gather(idx0, gidx0, poff0)
    fire_gathers(gidx0, buf0, gsem0)
    fire_idx(1, idx1, isem1)

    wait_idx(idx1, isem1)
    prep_gather(idx1, gidx1, poff1)
    fire_gathers(gidx1, buf1, gsem1)
    drain_gathers(gidx0, buf0, gsem0)
    fire_idx(2, idx0, isem0)
    compute(0, poff0, buf0, obuf0)
    fire_store(0, obuf0, ssem0)

    def pair(t, _):
        i = 2 * t + 1
        steady(i, s1, s0)
        steady(i + 1, s0, s1)
        return 0

    lax.fori_loop(0, (N_IT - 2) // 2, pair, 0)

    wait_store(obuf0, ssem0)
    drain_gathers(gidx1, buf1, gsem1)
    compute(N_IT - 1, poff1, buf1, obuf1)
    fire_store(N_IT - 1, obuf1, ssem1)
    wait_idx(idx0, isem0)
    wait_store(obuf1, ssem1)


@jax.jit
def kernel(token_ids, token_emb, pos_emb):
    mesh = plsc.VectorSubcoreMesh(core_axis_name="c", subcore_axis_name="s")
    params = pltpu.CompilerParams(
        use_tc_tiling_on_sc=False, needs_layout_passes=False
    )

    def slot_scratch():
        return [
            pltpu.VMEM((NS, BW), jnp.int32),
            pltpu.VMEM((NS, BW), jnp.int32),
            pltpu.VMEM((NS, BW), jnp.int32),
            pltpu.VMEM((NS, BW, 2 * EMBED), jnp.float32),
            pltpu.VMEM((NS, EMBED, OPITCH), jnp.float32),
        ]

    out = pl.kernel(
        _body,
        out_type=jax.ShapeDtypeStruct((SEQ, EMBED, BATCH), jnp.float32),
        mesh=mesh,
        compiler_params=params,
        scratch_types=[
            pltpu.VMEM((CTX * EMBED,), jnp.float32),
            *slot_scratch(),
            *slot_scratch(),
            pltpu.SemaphoreType.DMA,
            pltpu.SemaphoreType.DMA,
            pltpu.SemaphoreType.DMA,
            pltpu.SemaphoreType.DMA,
            pltpu.SemaphoreType.DMA,
            pltpu.SemaphoreType.DMA,
        ],
    )(
        token_ids.T.astype(jnp.int32),
        token_emb.reshape(VOCAB // 2, 2 * EMBED),
        pos_emb.reshape(CTX * EMBED),
    )
    return out.transpose(2, 0, 1)


# speedup vs baseline: 5.3235x; 1.0025x over previous
import functools

import jax
import jax.numpy as jnp
from jax import lax
from jax.experimental import pallas as pl
from jax.experimental.pallas import tpu as pltpu
from jax.experimental.pallas import tpu_sc as plsc

VOCAB = 1000000
EMBED = 64
CTX = 200
BATCH = 4096
SEQ = 200

N_WORKERS = 32
NK = EMBED // 16
BW = BATCH // N_WORKERS
NS = 2
N_IT = SEQ // NS
OPITCH = 133


def _worker_id():
    return lax.axis_index("s") * 2 + lax.axis_index("c")


def _body(idx_hbm, table_hbm, pos_hbm, out_hbm, pos_v,
          idx0, gidx0, poff0, buf0, obuf0,
          idx1, gidx1, poff1, buf1, obuf1,
          gsem0, gsem1, ssem0, ssem1, isem0, isem1):
    wid = _worker_id()
    bw0 = wid * BW

    pltpu.sync_copy(pos_hbm, pos_v)

    def fire_idx(i, idx, isem):
        s = jnp.minimum(i, N_IT - 1) * NS
        pltpu.async_copy(idx_hbm.at[pl.ds(s, NS), pl.ds(bw0, BW)], idx, isem)

    def wait_idx(idx, isem):
        pltpu.make_async_copy(
            idx_hbm.at[pl.ds(0, NS), pl.ds(bw0, BW)], idx, isem
        ).wait()

    def prep_gather(idx, gidx, poff):
        for j in range(NS):
            for c in range(BW // 16):
                sl = pl.ds(c * 16, 16)
                v = idx[j, sl]
                gidx[j, sl] = lax.shift_right_logical(v, 1)
                poff[j, sl] = lax.shift_left(v & 1, 6)

    def fire_gathers(gidx, buf, gsem):
        for j in range(NS):
            pltpu.async_copy(table_hbm.at[gidx.at[j]], buf.at[j], gsem)

    def drain_gathers(gidx, buf, gsem):
        for j in range(NS):
            pltpu.make_async_copy(table_hbm.at[gidx.at[j]], buf.at[j], gsem).wait()

    def compute(i, poff, buf, obuf):
        iota = lax.iota(jnp.int32, 16)
        drows = [iota + 16 * k for k in range(NK)]
        for j in range(NS):
            pbase = (i * NS + j) * EMBED
            pv = [pos_v[pl.ds(pbase + 16 * k, 16)] for k in range(NK)]
            jsplat = jnp.full((16,), j, jnp.int32)

            @plsc.parallel_loop(0, BW, 1, unroll=4)
            def _b(b):
                bsplat = jnp.full((16,), b, jnp.int32)
                pofb = plsc.load_gather(poff, [jsplat, bsplat])
                for k in range(NK):
                    v = plsc.load_gather(buf, [jsplat, bsplat, pofb + drows[k]])
                    plsc.store_scatter(obuf, [jsplat, drows[k], bsplat], v + pv[k])

    def fire_store(i, obuf, ssem):
        pltpu.async_copy(
            obuf.at[:, :, pl.ds(0, BW)],
            out_hbm.at[pl.ds(i * NS, NS), :, pl.ds(bw0, BW)],
            ssem,
        )

    def wait_store(obuf, ssem):
        pltpu.make_async_copy(
            obuf.at[:, :, pl.ds(0, BW)],
            out_hbm.at[pl.ds(0, NS), :, pl.ds(bw0, BW)],
            ssem,
        ).wait()

    s0 = (idx0, gidx0, poff0, buf0, obuf0, gsem0, ssem0, isem0)
    s1 = (idx1, gidx1, poff1, buf1, obuf1, gsem1, ssem1, isem1)

    def steady(i, X, Y):
        (idxX, gidxX, poffX, bufX, obufX, gsemX, ssemX, isemX) = X
        (idxY, gidxY, poffY, bufY, obufY, gsemY, ssemY, isemY) = Y
        wait_store(obufY, ssemY)
        wait_idx(idxY, isemY)
        prep_gather(idxY, gidxY, poffY)
        fire_gathers(gidxY, bufY, gsemY)
        drain_gathers(gidxX, bufX, gsemX)
        fire_idx(i + 2, idxX, isemX)
        compute(i, poffX, bufX, obufX)
        fire_store(i, obufX, ssemX)

    fire_idx(0, idx0, isem0)
    wait_idx(idx0, isem0)
    prep_gather(idx0, gidx0, poff0)
    fire_gathers(gidx0, buf0, gsem0)
    fire_idx(1, idx1, isem1)

    wait_idx(idx1, isem1)
    prep_gather(idx1, gidx1, poff1)
    fire_gathers(gidx1, buf1, gsem1)
    drain_gathers(gidx0, buf0, gsem0)
    fire_idx(2, idx0, isem0)
    compute(0, poff0, buf0, obuf0)
    fire_store(0, obuf0, ssem0)

    def pair(t, _):
        i = 2 * t + 1
        steady(i, s1, s0)
        steady(i + 1, s0, s1)
        return 0

    lax.fori_loop(0, (N_IT - 2) // 2, pair, 0)

    wait_store(obuf0, ssem0)
    drain_gathers(gidx1, buf1, gsem1)
    compute(N_IT - 1, poff1, buf1, obuf1)
    fire_store(N_IT - 1, obuf1, ssem1)
    wait_idx(idx0, isem0)
    wait_store(obuf1, ssem1)


@jax.jit
def kernel(token_ids, token_emb, pos_emb):
    mesh = plsc.VectorSubcoreMesh(core_axis_name="c", subcore_axis_name="s")
    params = pltpu.CompilerParams(
        use_tc_tiling_on_sc=False, needs_layout_passes=False
    )

    def slot_scratch():
        return [
            pltpu.VMEM((NS, BW), jnp.int32),
            pltpu.VMEM((NS, BW), jnp.int32),
            pltpu.VMEM((NS, BW), jnp.int32),
            pltpu.VMEM((NS, BW, 2 * EMBED), jnp.float32),
            pltpu.VMEM((NS, EMBED, OPITCH), jnp.float32),
        ]

    out = pl.kernel(
        _body,
        out_type=jax.ShapeDtypeStruct((SEQ, EMBED, BATCH), jnp.float32),
        mesh=mesh,
        compiler_params=params,
        scratch_types=[
            pltpu.VMEM((CTX * EMBED,), jnp.float32),
            *slot_scratch(),
            *slot_scratch(),
            pltpu.SemaphoreType.DMA,
            pltpu.SemaphoreType.DMA,
            pltpu.SemaphoreType.DMA,
            pltpu.SemaphoreType.DMA,
            pltpu.SemaphoreType.DMA,
            pltpu.SemaphoreType.DMA,
        ],
    )(
        token_ids.T.astype(jnp.int32),
        token_emb.reshape(VOCAB // 2, 2 * EMBED),
        pos_emb.reshape(CTX * EMBED),
    )
    return out.transpose(2, 0, 1)
